# baseline (speedup 1.0000x reference)
import jax
import jax.numpy as jnp
from jax import lax
from jax.experimental import pallas as pl
from jax.experimental.pallas import tpu as pltpu
from jax.experimental.pallas import tpu_sc as plsc

_M = 10000
_B = 4096
_D = 64
_P = _B * (_B - 1) // 2
_NW = 32
_RP_PER_W = (_B // 2) // _NW
_NG = _B // 16
_CAP = (_NG + 1) * 16
_LANES = 16


_MAXCH = 2097152
_NDM = _M * (_M - 1) // 2


def _max_body(pos_ref, x_ref, o_ref):
    g = pl.program_id(0)

    @pl.when(g == 0)
    def _():
        o_ref[0, 0] = jnp.float32(0.0)

    pos = pos_ref[...] + g * _MAXCH
    x = jnp.where(pos < _NDM, x_ref[...], 0.0)
    o_ref[0, 0] = jnp.maximum(o_ref[0, 0], jnp.max(x))


def _dmat_max(dmat):
    grid = (_NDM + _MAXCH - 1) // _MAXCH
    pos = jnp.arange(_MAXCH, dtype=jnp.int32)
    return pl.pallas_call(
        _max_body,
        grid=(grid,),
        in_specs=[
            pl.BlockSpec((_MAXCH,), lambda g: (0,)),
            pl.BlockSpec((_MAXCH,), lambda g: (g,)),
        ],
        out_specs=pl.BlockSpec(memory_space=pltpu.SMEM),
        out_shape=jax.ShapeDtypeStruct((1, 1), jnp.float32),
    )(pos, dmat)


_TILE = 512


_TILEJ = 2048


def _dist_body(dmax_ref, xi_ref, xj_ref, ones_ref, o_ref):
    xi = xi_ref[...]
    xj = xj_ref[...]
    g = lax.dot_general(
        xi, xj, (((1,), (1,)), ((), ())),
        preferred_element_type=jnp.float32,
        precision=lax.Precision.HIGHEST,
    )
    sqi = jnp.sum(xi * xi, axis=1, keepdims=True)
    sqj = lax.dot_general(
        ones_ref[...], xj * xj, (((1,), (1,)), ((), ())),
        preferred_element_type=jnp.float32,
        precision=lax.Precision.HIGHEST,
    )
    d2 = sqi + sqj - 2.0 * g
    o_ref[...] = jnp.sqrt(jnp.maximum(d2, 0.0)) * dmax_ref[0, 0]


def _dist_matrix(output, dmax):
    ones = jnp.ones((1, _D), jnp.float32)
    return pl.pallas_call(
        _dist_body,
        grid=(_B // _TILE, _B // _TILEJ),
        in_specs=[
            pl.BlockSpec(memory_space=pltpu.SMEM),
            pl.BlockSpec((_TILE, _D), lambda i, j: (i, 0)),
            pl.BlockSpec((_TILEJ, _D), lambda i, j: (j, 0)),
            pl.BlockSpec((1, _D), lambda i, j: (0, 0)),
        ],
        out_specs=pl.BlockSpec((_TILE, _TILEJ), lambda i, j: (i, j)),
        out_shape=jax.ShapeDtypeStruct((_B, _B), jnp.float32),
    )(dmax, output, output, ones)


def _sc_body(tc_hbm, d_hbm, dmat_hbm, out_hbm,
             tcv,
             cidx_a, mask_a, tval_a, drow1_a, drow2_a,
             cidx_b, mask_b, tval_b, drow1_b, drow2_b,
             accv,
             sem_d1a, sem_d2a, sem_ga, sem_d1b, sem_d2b, sem_gb):
    wid = lax.axis_index("s") * 2 + lax.axis_index("c")
    row0 = wid * _RP_PER_W
    iota = lax.broadcasted_iota(jnp.int32, (_LANES,), 0)

    buf_a = (cidx_a, mask_a, tval_a, drow1_a, drow2_a,
             sem_d1a, sem_d2a, sem_ga)
    buf_b = (cidx_b, mask_b, tval_b, drow1_b, drow2_b,
             sem_d1b, sem_d2b, sem_gb)

    pltpu.sync_copy(tc_hbm, tcv)
    accv[...] = jnp.zeros((_LANES,), jnp.float32)

    def zero_body(k, _):
        z = jnp.zeros((_LANES,), jnp.int32)
        cidx_a[pl.ds(k * _LANES, _LANES)] = z
        cidx_b[pl.ds(k * _LANES, _LANES)] = z
        return ()

    lax.fori_loop(0, _CAP // _LANES, zero_body, ())

    def row_geom(t):
        r1 = row0 + t
        r2 = (_B - 1) - r1
        g1 = lax.shift_right_logical(r1 + 1, 4)
        g2 = lax.shift_right_logical(r2 + 1, 4)
        return r1, r2, g1, g2

    def splat_tc(r):
        idx = jnp.full((_LANES,), r, jnp.int32)
        return plsc.load_gather(tcv, [idx])

    def cond_idx(ta16, tc16):
        i16 = jnp.minimum(ta16, tc16)
        j16 = jnp.maximum(ta16, tc16)
        neq = jnp.not_equal(i16, j16)
        ci = _M * i16 + j16 - lax.shift_right_logical(
            (i16 + 2) * (i16 + 1), 1)
        return jnp.where(neq, ci, 0), neq

    def stage1(t, buf):
        cidx_v, mask_v, tval_v, drow1, drow2, sem_d1, sem_d2, sem_g = buf
        r1, r2, g1, g2 = row_geom(t)
        pltpu.make_async_copy(d_hbm.at[r1], drow1, sem_d1).start()
        pltpu.make_async_copy(d_hbm.at[r2], drow2, sem_d2).start()

        ta1 = splat_tc(r1)
        ta2 = splat_tc(r2)
        o2 = (_NG - g1) * _LANES

        bvec = iota + g1 * _LANES
        tc16 = tcv[pl.ds(g1 * _LANES, _LANES)]
        ci, neq = cond_idx(ta1, tc16)
        m = jnp.logical_and(neq, bvec > r1)
        cidx_v[pl.ds(0, _LANES)] = ci
        mask_v[pl.ds(0, _LANES)] = jnp.where(m, 1.0, 0.0).astype(jnp.float32)

        def loop1(k, _):
            tck = tcv[pl.ds(k * _LANES, _LANES)]
            cik, neqk = cond_idx(ta1, tck)
            slot = (k - g1) * _LANES
            cidx_v[pl.ds(slot, _LANES)] = cik
            mask_v[pl.ds(slot, _LANES)] = jnp.where(
                neqk, 1.0, 0.0).astype(jnp.float32)
            return ()

        lax.fori_loop(g1 + 1, _NG, loop1, ())

        @pl.when(g2 < _NG)
        def _():
            bvec2 = iota + g2 * _LANES
            tcg = tcv[pl.ds(g2 * _LANES, _LANES)]
            ci2, neq2 = cond_idx(ta2, tcg)
            m2 = jnp.logical_and(neq2, bvec2 > r2)
            cidx_v[pl.ds(o2, _LANES)] = ci2
            mask_v[pl.ds(o2, _LANES)] = jnp.where(
                m2, 1.0, 0.0).astype(jnp.float32)

        def loop2(k, _):
            tck = tcv[pl.ds(k * _LANES, _LANES)]
            cik, neqk = cond_idx(ta2, tck)
            slot = o2 + (k - g2) * _LANES
            cidx_v[pl.ds(slot, _LANES)] = cik
            mask_v[pl.ds(slot, _LANES)] = jnp.where(
                neqk, 1.0, 0.0).astype(jnp.float32)
            return ()

        lax.fori_loop(jnp.minimum(g2 + 1, _NG), _NG, loop2, ())

        pltpu.make_async_copy(dmat_hbm.at[cidx_v], tval_v, sem_g).start()

    def stage2(t, buf):
        cidx_v, mask_v, tval_v, drow1, drow2, sem_d1, sem_d2, sem_g = buf
        r1, r2, g1, g2 = row_geom(t)
        o2 = (_NG - g1) * _LANES
        pltpu.make_async_copy(d_hbm.at[r1], drow1, sem_d1).wait()
        pltpu.make_async_copy(d_hbm.at[r2], drow2, sem_d2).wait()

        bvec = iota + g1 * _LANES
        d16 = drow1[pl.ds(g1 * _LANES, _LANES)]
        drow1[pl.ds(g1 * _LANES, _LANES)] = jnp.where(bvec > r1, d16, 0.0)

        @pl.when(g2 < _NG)
        def _():
            bvec2 = iota + g2 * _LANES
            d16b = drow2[pl.ds(g2 * _LANES, _LANES)]
            drow2[pl.ds(g2 * _LANES, _LANES)] = jnp.where(
                bvec2 > r2, d16b, 0.0)

        pltpu.make_async_copy(dmat_hbm.at[cidx_v], tval_v, sem_g).wait()

        def acc1(k, s):
            d = drow1[pl.ds(k * _LANES, _LANES)]
            slot = (k - g1) * _LANES
            tv = tval_v[pl.ds(slot, _LANES)]
            mv = mask_v[pl.ds(slot, _LANES)]
            return s + jnp.abs(d - tv * mv)

        s1 = lax.fori_loop(g1, _NG, acc1,
                           jnp.zeros((_LANES,), jnp.float32))

        def acc2(k, s):
            d = drow2[pl.ds(k * _LANES, _LANES)]
            slot = o2 + (k - g2) * _LANES
            tv = tval_v[pl.ds(slot, _LANES)]
            mv = mask_v[pl.ds(slot, _LANES)]
            return s + jnp.abs(d - tv * mv)

        s2 = lax.fori_loop(jnp.minimum(g2, _NG), _NG, acc2, s1)
        accv[...] = accv[...] + s2

    stage1(jnp.int32(0), buf_a)

    def pipe(u, _):
        t1 = 2 * u + 1
        t2 = 2 * u + 2
        stage1(t1, buf_b)
        stage2(t1 - 1, buf_a)

        @pl.when(t2 < _RP_PER_W)
        def _():
            stage1(t2, buf_a)

        stage2(t1, buf_b)
        return ()

    lax.fori_loop(0, _RP_PER_W // 2, pipe, ())

    pltpu.sync_copy(accv, out_hbm.at[wid])


def _sc_pair_loss(target_cls, dist, dmat):
    mesh = plsc.VectorSubcoreMesh(
        core_axis_name="c", subcore_axis_name="s", num_cores=2,
        num_subcores=16)
    f = pl.kernel(
        _sc_body,
        out_type=jax.ShapeDtypeStruct((_NW, _LANES), jnp.float32),
        mesh=mesh,
        compiler_params=pltpu.CompilerParams(needs_layout_passes=False),
        scratch_types=[
            pltpu.VMEM((_B,), jnp.int32),
            pltpu.VMEM((_CAP,), jnp.int32),
            pltpu.VMEM((_CAP,), jnp.float32),
            pltpu.VMEM((_CAP,), jnp.float32),
            pltpu.VMEM((_B,), jnp.float32),
            pltpu.VMEM((_B,), jnp.float32),
            pltpu.VMEM((_CAP,), jnp.int32),
            pltpu.VMEM((_CAP,), jnp.float32),
            pltpu.VMEM((_CAP,), jnp.float32),
            pltpu.VMEM((_B,), jnp.float32),
            pltpu.VMEM((_B,), jnp.float32),
            pltpu.VMEM((_LANES,), jnp.float32),
            pltpu.SemaphoreType.DMA,
            pltpu.SemaphoreType.DMA,
            pltpu.SemaphoreType.DMA,
            pltpu.SemaphoreType.DMA,
            pltpu.SemaphoreType.DMA,
            pltpu.SemaphoreType.DMA,
        ],
    )
    return f(target_cls, dist, dmat)


def kernel(output, target_cls, dmat):
    dmax = _dmat_max(dmat)
    dist = _dist_matrix(output, dmax)
    partials = _sc_pair_loss(target_cls, dist, dmat)
    return jnp.sum(partials) / (jnp.float32(_P) * dmax[0, 0])


# speedup vs baseline: 1.2137x; 1.2137x over previous
import jax
import jax.numpy as jnp
from jax import lax
from jax.experimental import pallas as pl
from jax.experimental.pallas import tpu as pltpu
from jax.experimental.pallas import tpu_sc as plsc

_M = 10000
_B = 4096
_D = 64
_P = _B * (_B - 1) // 2
_NW = 32
_RP_PER_W = (_B // 2) // _NW
_NG = _B // 16
_CAP = (_NG + 1) * 16
_LANES = 16


_NDM = _M * (_M - 1) // 2
_MAXCH = 4864 * 1024
_MAXG = _NDM // _MAXCH
_MAXTAIL = _NDM - _MAXG * _MAXCH


def _max_body(x_ref, tail_ref, o_ref):
    g = pl.program_id(0)

    @pl.when(g == 0)
    def _():
        o_ref[0, 0] = jnp.max(tail_ref[...])

    o_ref[0, 0] = jnp.maximum(o_ref[0, 0], jnp.max(x_ref[...]))


def _dmat_max(dmat):
    tail = lax.slice(dmat, (_MAXG * _MAXCH,), (_NDM,))
    return pl.pallas_call(
        _max_body,
        grid=(_MAXG,),
        in_specs=[
            pl.BlockSpec((_MAXCH,), lambda g: (g,)),
            pl.BlockSpec((_MAXTAIL,), lambda g: (0,)),
        ],
        out_specs=pl.BlockSpec(memory_space=pltpu.SMEM),
        out_shape=jax.ShapeDtypeStruct((1, 1), jnp.float32),
    )(dmat, tail)


_TILE = 512


def _sq_body(x_ref, ones_ref, sqc_ref, sqr_ref):
    xx = x_ref[...] * x_ref[...]
    sqc_ref[...] = jnp.sum(xx, axis=1, keepdims=True)
    sqr_ref[...] = lax.dot_general(
        ones_ref[...], xx, (((1,), (1,)), ((), ())),
        preferred_element_type=jnp.float32,
        precision=lax.Precision.HIGHEST,
    )


def _sq_norms(output):
    ones = jnp.ones((1, _D), jnp.float32)
    return pl.pallas_call(
        _sq_body,
        out_shape=(
            jax.ShapeDtypeStruct((_B, 1), jnp.float32),
            jax.ShapeDtypeStruct((1, _B), jnp.float32),
        ),
    )(output, ones)


def _dist_body(dmax_ref, xi_ref, xj_ref, sqc_ref, sqr_ref, o_ref):
    g = lax.dot_general(
        xi_ref[...], xj_ref[...], (((1,), (1,)), ((), ())),
        preferred_element_type=jnp.float32,
        precision=lax.Precision.HIGHEST,
    )
    d2 = sqc_ref[...] + sqr_ref[...] - 2.0 * g
    o_ref[...] = jnp.sqrt(jnp.maximum(d2, 0.0)) * dmax_ref[0, 0]


def _dist_matrix(output, dmax):
    sqc, sqr = _sq_norms(output)
    nt = _B // _TILE
    return pl.pallas_call(
        _dist_body,
        grid=(nt, nt),
        in_specs=[
            pl.BlockSpec(memory_space=pltpu.SMEM),
            pl.BlockSpec((_TILE, _D), lambda i, j: (i, 0)),
            pl.BlockSpec((_TILE, _D), lambda i, j: (j, 0)),
            pl.BlockSpec((_TILE, 1), lambda i, j: (i, 0)),
            pl.BlockSpec((1, _TILE), lambda i, j: (0, j)),
        ],
        out_specs=pl.BlockSpec((_TILE, _TILE), lambda i, j: (i, j)),
        out_shape=jax.ShapeDtypeStruct((_B, _B), jnp.float32),
    )(dmax, output, output, sqc, sqr)


def _sc_body(tc_hbm, d_hbm, dmat_hbm, out_hbm,
             tcv,
             cidx_a, mask_a, tval_a, drow1_a, drow2_a,
             cidx_b, mask_b, tval_b, drow1_b, drow2_b,
             accv,
             sem_d1a, sem_d2a, sem_ga, sem_d1b, sem_d2b, sem_gb):
    wid = lax.axis_index("s") * 2 + lax.axis_index("c")
    row0 = wid * _RP_PER_W
    iota = lax.broadcasted_iota(jnp.int32, (_LANES,), 0)

    buf_a = (cidx_a, mask_a, tval_a, drow1_a, drow2_a,
             sem_d1a, sem_d2a, sem_ga)
    buf_b = (cidx_b, mask_b, tval_b, drow1_b, drow2_b,
             sem_d1b, sem_d2b, sem_gb)

    pltpu.sync_copy(tc_hbm, tcv)
    accv[...] = jnp.zeros((_LANES,), jnp.float32)

    def zero_body(k, _):
        z = jnp.zeros((_LANES,), jnp.int32)
        cidx_a[pl.ds(k * _LANES, _LANES)] = z
        cidx_b[pl.ds(k * _LANES, _LANES)] = z
        return ()

    lax.fori_loop(0, _CAP // _LANES, zero_body, ())

    def row_geom(t):
        r1 = row0 + t
        r2 = (_B - 1) - r1
        g1 = lax.shift_right_logical(r1 + 1, 4)
        g2 = lax.shift_right_logical(r2 + 1, 4)
        return r1, r2, g1, g2

    def splat_tc(r):
        idx = jnp.full((_LANES,), r, jnp.int32)
        return plsc.load_gather(tcv, [idx])

    def cond_idx(ta16, tc16):
        i16 = jnp.minimum(ta16, tc16)
        j16 = jnp.maximum(ta16, tc16)
        neq = jnp.not_equal(i16, j16)
        ci = _M * i16 + j16 - lax.shift_right_logical(
            (i16 + 2) * (i16 + 1), 1)
        return jnp.where(neq, ci, 0), neq

    def stage1(t, buf):
        cidx_v, mask_v, tval_v, drow1, drow2, sem_d1, sem_d2, sem_g = buf
        r1, r2, g1, g2 = row_geom(t)
        pltpu.make_async_copy(d_hbm.at[r1], drow1, sem_d1).start()
        pltpu.make_async_copy(d_hbm.at[r2], drow2, sem_d2).start()

        ta1 = splat_tc(r1)
        ta2 = splat_tc(r2)
        o2 = (_NG - g1) * _LANES

        bvec = iota + g1 * _LANES
        tc16 = tcv[pl.ds(g1 * _LANES, _LANES)]
        ci, neq = cond_idx(ta1, tc16)
        m = jnp.logical_and(neq, bvec > r1)
        cidx_v[pl.ds(0, _LANES)] = ci
        mask_v[pl.ds(0, _LANES)] = jnp.where(m, 1.0, 0.0).astype(jnp.float32)

        def loop1(k, _):
            tck = tcv[pl.ds(k * _LANES, _LANES)]
            cik, neqk = cond_idx(ta1, tck)
            slot = (k - g1) * _LANES
            cidx_v[pl.ds(slot, _LANES)] = cik
            mask_v[pl.ds(slot, _LANES)] = jnp.where(
                neqk, 1.0, 0.0).astype(jnp.float32)
            return ()

        lax.fori_loop(g1 + 1, _NG, loop1, ())

        @pl.when(g2 < _NG)
        def _():
            bvec2 = iota + g2 * _LANES
            tcg = tcv[pl.ds(g2 * _LANES, _LANES)]
            ci2, neq2 = cond_idx(ta2, tcg)
            m2 = jnp.logical_and(neq2, bvec2 > r2)
            cidx_v[pl.ds(o2, _LANES)] = ci2
            mask_v[pl.ds(o2, _LANES)] = jnp.where(
                m2, 1.0, 0.0).astype(jnp.float32)

        def loop2(k, _):
            tck = tcv[pl.ds(k * _LANES, _LANES)]
            cik, neqk = cond_idx(ta2, tck)
            slot = o2 + (k - g2) * _LANES
            cidx_v[pl.ds(slot, _LANES)] = cik
            mask_v[pl.ds(slot, _LANES)] = jnp.where(
                neqk, 1.0, 0.0).astype(jnp.float32)
            return ()

        lax.fori_loop(jnp.minimum(g2 + 1, _NG), _NG, loop2, ())

        pltpu.make_async_copy(dmat_hbm.at[cidx_v], tval_v, sem_g).start()

    def stage2(t, buf):
        cidx_v, mask_v, tval_v, drow1, drow2, sem_d1, sem_d2, sem_g = buf
        r1, r2, g1, g2 = row_geom(t)
        o2 = (_NG - g1) * _LANES
        pltpu.make_async_copy(d_hbm.at[r1], drow1, sem_d1).wait()
        pltpu.make_async_copy(d_hbm.at[r2], drow2, sem_d2).wait()

        bvec = iota + g1 * _LANES
        d16 = drow1[pl.ds(g1 * _LANES, _LANES)]
        drow1[pl.ds(g1 * _LANES, _LANES)] = jnp.where(bvec > r1, d16, 0.0)

        @pl.when(g2 < _NG)
        def _():
            bvec2 = iota + g2 * _LANES
            d16b = drow2[pl.ds(g2 * _LANES, _LANES)]
            drow2[pl.ds(g2 * _LANES, _LANES)] = jnp.where(
                bvec2 > r2, d16b, 0.0)

        pltpu.make_async_copy(dmat_hbm.at[cidx_v], tval_v, sem_g).wait()

        def acc1(k, s):
            d = drow1[pl.ds(k * _LANES, _LANES)]
            slot = (k - g1) * _LANES
            tv = tval_v[pl.ds(slot, _LANES)]
            mv = mask_v[pl.ds(slot, _LANES)]
            return s + jnp.abs(d - tv * mv)

        s1 = lax.fori_loop(g1, _NG, acc1,
                           jnp.zeros((_LANES,), jnp.float32))

        def acc2(k, s):
            d = drow2[pl.ds(k * _LANES, _LANES)]
            slot = o2 + (k - g2) * _LANES
            tv = tval_v[pl.ds(slot, _LANES)]
            mv = mask_v[pl.ds(slot, _LANES)]
            return s + jnp.abs(d - tv * mv)

        s2 = lax.fori_loop(jnp.minimum(g2, _NG), _NG, acc2, s1)
        accv[...] = accv[...] + s2

    stage1(jnp.int32(0), buf_a)

    def pipe(u, _):
        t1 = 2 * u + 1
        t2 = 2 * u + 2
        stage1(t1, buf_b)
        stage2(t1 - 1, buf_a)

        @pl.when(t2 < _RP_PER_W)
        def _():
            stage1(t2, buf_a)

        stage2(t1, buf_b)
        return ()

    lax.fori_loop(0, _RP_PER_W // 2, pipe, ())

    pltpu.sync_copy(accv, out_hbm.at[wid])


def _sc_pair_loss(target_cls, dist, dmat):
    mesh = plsc.VectorSubcoreMesh(
        core_axis_name="c", subcore_axis_name="s", num_cores=2,
        num_subcores=16)
    f = pl.kernel(
        _sc_body,
        out_type=jax.ShapeDtypeStruct((_NW, _LANES), jnp.float32),
        mesh=mesh,
        compiler_params=pltpu.CompilerParams(needs_layout_passes=False),
        scratch_types=[
            pltpu.VMEM((_B,), jnp.int32),
            pltpu.VMEM((_CAP,), jnp.int32),
            pltpu.VMEM((_CAP,), jnp.float32),
            pltpu.VMEM((_CAP,), jnp.float32),
            pltpu.VMEM((_B,), jnp.float32),
            pltpu.VMEM((_B,), jnp.float32),
            pltpu.VMEM((_CAP,), jnp.int32),
            pltpu.VMEM((_CAP,), jnp.float32),
            pltpu.VMEM((_CAP,), jnp.float32),
            pltpu.VMEM((_B,), jnp.float32),
            pltpu.VMEM((_B,), jnp.float32),
            pltpu.VMEM((_LANES,), jnp.float32),
            pltpu.SemaphoreType.DMA,
            pltpu.SemaphoreType.DMA,
            pltpu.SemaphoreType.DMA,
            pltpu.SemaphoreType.DMA,
            pltpu.SemaphoreType.DMA,
            pltpu.SemaphoreType.DMA,
        ],
    )
    return f(target_cls, dist, dmat)


def kernel(output, target_cls, dmat):
    dmax = _dmat_max(dmat)
    dist = _dist_matrix(output, dmax)
    partials = _sc_pair_loss(target_cls, dist, dmat)
    return jnp.sum(partials) / (jnp.float32(_P) * dmax[0, 0])


# speedup vs baseline: 1.2767x; 1.0520x over previous
import jax
import jax.numpy as jnp
from jax import lax
from jax.experimental import pallas as pl
from jax.experimental.pallas import tpu as pltpu
from jax.experimental.pallas import tpu_sc as plsc

_M = 10000
_B = 4096
_D = 64
_P = _B * (_B - 1) // 2
_NW = 32
_RP_PER_W = (_B // 2) // _NW
_NG = _B // 16
_CAP = (_NG + 1) * 16
_LANES = 16


_NDM = _M * (_M - 1) // 2
_MAXCH = 4864 * 1024
_MAXG = _NDM // _MAXCH
_MAXTAIL = _NDM - _MAXG * _MAXCH


def _max_body(x_ref, tail_ref, o_ref):
    g = pl.program_id(0)

    @pl.when(g == 0)
    def _():
        o_ref[0, 0] = jnp.max(tail_ref[...])

    o_ref[0, 0] = jnp.maximum(o_ref[0, 0], jnp.max(x_ref[...]))


def _dmat_max(dmat):
    tail = lax.slice(dmat, (_MAXG * _MAXCH,), (_NDM,))
    return pl.pallas_call(
        _max_body,
        grid=(_MAXG,),
        in_specs=[
            pl.BlockSpec((_MAXCH,), lambda g: (g,)),
            pl.BlockSpec((_MAXTAIL,), lambda g: (0,)),
        ],
        out_specs=pl.BlockSpec(memory_space=pltpu.SMEM),
        out_shape=jax.ShapeDtypeStruct((1, 1), jnp.float32),
    )(dmat, tail)


_TILE = 512


def _sq_body(x_ref, ones_ref, sqc_ref, sqr_ref):
    xx = x_ref[...] * x_ref[...]
    sqc_ref[...] = jnp.sum(xx, axis=1, keepdims=True)
    sqr_ref[...] = lax.dot_general(
        ones_ref[...], xx, (((1,), (1,)), ((), ())),
        preferred_element_type=jnp.float32,
        precision=lax.Precision.HIGHEST,
    )


def _sq_norms(output):
    ones = jnp.ones((1, _D), jnp.float32)
    return pl.pallas_call(
        _sq_body,
        out_shape=(
            jax.ShapeDtypeStruct((_B, 1), jnp.float32),
            jax.ShapeDtypeStruct((1, _B), jnp.float32),
        ),
    )(output, ones)


def _dist_body(dmax_ref, xi_ref, xj_ref, sqc_ref, sqr_ref, o_ref):
    g = lax.dot_general(
        xi_ref[...], xj_ref[...], (((1,), (1,)), ((), ())),
        preferred_element_type=jnp.float32,
        precision=lax.Precision.DEFAULT,
    )
    d2 = sqc_ref[...] + sqr_ref[...] - 2.0 * g
    o_ref[...] = jnp.sqrt(jnp.maximum(d2, 0.0)) * dmax_ref[0, 0]


def _dist_matrix(output, dmax):
    sqc, sqr = _sq_norms(output)
    nt = _B // _TILE
    return pl.pallas_call(
        _dist_body,
        grid=(nt, nt),
        in_specs=[
            pl.BlockSpec(memory_space=pltpu.SMEM),
            pl.BlockSpec((_TILE, _D), lambda i, j: (i, 0)),
            pl.BlockSpec((_TILE, _D), lambda i, j: (j, 0)),
            pl.BlockSpec((_TILE, 1), lambda i, j: (i, 0)),
            pl.BlockSpec((1, _TILE), lambda i, j: (0, j)),
        ],
        out_specs=pl.BlockSpec((_TILE, _TILE), lambda i, j: (i, j)),
        out_shape=jax.ShapeDtypeStruct((_B, _B), jnp.float32),
    )(dmax, output, output, sqc, sqr)


def _sc_body(tc_hbm, d_hbm, dmat_hbm, out_hbm,
             tcv,
             cidx_a, mask_a, tval_a, drow1_a, drow2_a,
             cidx_b, mask_b, tval_b, drow1_b, drow2_b,
             accv,
             sem_d1a, sem_d2a, sem_ga, sem_d1b, sem_d2b, sem_gb):
    wid = lax.axis_index("s") * 2 + lax.axis_index("c")
    row0 = wid * _RP_PER_W
    iota = lax.broadcasted_iota(jnp.int32, (_LANES,), 0)

    buf_a = (cidx_a, mask_a, tval_a, drow1_a, drow2_a,
             sem_d1a, sem_d2a, sem_ga)
    buf_b = (cidx_b, mask_b, tval_b, drow1_b, drow2_b,
             sem_d1b, sem_d2b, sem_gb)

    pltpu.sync_copy(tc_hbm, tcv)
    accv[...] = jnp.zeros((_LANES,), jnp.float32)

    def zero_body(k, _):
        z = jnp.zeros((_LANES,), jnp.int32)
        cidx_a[pl.ds(k * _LANES, _LANES)] = z
        cidx_b[pl.ds(k * _LANES, _LANES)] = z
        return ()

    lax.fori_loop(0, _CAP // _LANES, zero_body, ())

    def row_geom(t):
        r1 = row0 + t
        r2 = (_B - 1) - r1
        g1 = lax.shift_right_logical(r1 + 1, 4)
        g2 = lax.shift_right_logical(r2 + 1, 4)
        return r1, r2, g1, g2

    def splat_tc(r):
        idx = jnp.full((_LANES,), r, jnp.int32)
        return plsc.load_gather(tcv, [idx])

    def cond_idx(ta16, tc16):
        i16 = jnp.minimum(ta16, tc16)
        j16 = jnp.maximum(ta16, tc16)
        neq = jnp.not_equal(i16, j16)
        ci = _M * i16 + j16 - lax.shift_right_logical(
            (i16 + 2) * (i16 + 1), 1)
        return jnp.where(neq, ci, 0), neq

    def stage1(t, buf):
        cidx_v, mask_v, tval_v, drow1, drow2, sem_d1, sem_d2, sem_g = buf
        r1, r2, g1, g2 = row_geom(t)
        pltpu.make_async_copy(d_hbm.at[r1], drow1, sem_d1).start()
        pltpu.make_async_copy(d_hbm.at[r2, pl.ds(_B // 2, _B // 2)], drow2,
                              sem_d2).start()

        ta1 = splat_tc(r1)
        ta2 = splat_tc(r2)
        o2 = (_NG - g1) * _LANES

        bvec = iota + g1 * _LANES
        tc16 = tcv[pl.ds(g1 * _LANES, _LANES)]
        ci, neq = cond_idx(ta1, tc16)
        m = jnp.logical_and(neq, bvec > r1)
        cidx_v[pl.ds(0, _LANES)] = ci
        mask_v[pl.ds(0, _LANES)] = jnp.where(m, 1.0, 0.0).astype(jnp.float32)

        def loop1(k, _):
            tck = tcv[pl.ds(k * _LANES, _LANES)]
            cik, neqk = cond_idx(ta1, tck)
            slot = (k - g1) * _LANES
            cidx_v[pl.ds(slot, _LANES)] = cik
            mask_v[pl.ds(slot, _LANES)] = jnp.where(
                neqk, 1.0, 0.0).astype(jnp.float32)
            return ()

        lax.fori_loop(g1 + 1, _NG, loop1, ())

        @pl.when(g2 < _NG)
        def _():
            bvec2 = iota + g2 * _LANES
            tcg = tcv[pl.ds(g2 * _LANES, _LANES)]
            ci2, neq2 = cond_idx(ta2, tcg)
            m2 = jnp.logical_and(neq2, bvec2 > r2)
            cidx_v[pl.ds(o2, _LANES)] = ci2
            mask_v[pl.ds(o2, _LANES)] = jnp.where(
                m2, 1.0, 0.0).astype(jnp.float32)

        def loop2(k, _):
            tck = tcv[pl.ds(k * _LANES, _LANES)]
            cik, neqk = cond_idx(ta2, tck)
            slot = o2 + (k - g2) * _LANES
            cidx_v[pl.ds(slot, _LANES)] = cik
            mask_v[pl.ds(slot, _LANES)] = jnp.where(
                neqk, 1.0, 0.0).astype(jnp.float32)
            return ()

        lax.fori_loop(jnp.minimum(g2 + 1, _NG), _NG, loop2, ())

        pltpu.make_async_copy(dmat_hbm.at[cidx_v], tval_v, sem_g).start()

    def stage2(t, buf):
        cidx_v, mask_v, tval_v, drow1, drow2, sem_d1, sem_d2, sem_g = buf
        r1, r2, g1, g2 = row_geom(t)
        o2 = (_NG - g1) * _LANES
        pltpu.make_async_copy(d_hbm.at[r1], drow1, sem_d1).wait()
        pltpu.make_async_copy(d_hbm.at[r2, pl.ds(_B // 2, _B // 2)], drow2,
                              sem_d2).wait()

        bvec = iota + g1 * _LANES
        d16 = drow1[pl.ds(g1 * _LANES, _LANES)]
        drow1[pl.ds(g1 * _LANES, _LANES)] = jnp.where(bvec > r1, d16, 0.0)

        @pl.when(g2 < _NG)
        def _():
            bvec2 = iota + g2 * _LANES
            off2 = g2 * _LANES - _B // 2
            d16b = drow2[pl.ds(off2, _LANES)]
            drow2[pl.ds(off2, _LANES)] = jnp.where(
                bvec2 > r2, d16b, 0.0)

        pltpu.make_async_copy(dmat_hbm.at[cidx_v], tval_v, sem_g).wait()

        def acc1(k, s):
            d = drow1[pl.ds(k * _LANES, _LANES)]
            slot = (k - g1) * _LANES
            tv = tval_v[pl.ds(slot, _LANES)]
            mv = mask_v[pl.ds(slot, _LANES)]
            return s + jnp.abs(d - tv * mv)

        s1 = lax.fori_loop(g1, _NG, acc1,
                           jnp.zeros((_LANES,), jnp.float32))

        def acc2(k, s):
            d = drow2[pl.ds(k * _LANES - _B // 2, _LANES)]
            slot = o2 + (k - g2) * _LANES
            tv = tval_v[pl.ds(slot, _LANES)]
            mv = mask_v[pl.ds(slot, _LANES)]
            return s + jnp.abs(d - tv * mv)

        s2 = lax.fori_loop(jnp.minimum(g2, _NG), _NG, acc2, s1)
        accv[...] = accv[...] + s2

    stage1(jnp.int32(0), buf_a)

    def pipe(u, _):
        t1 = 2 * u + 1
        t2 = 2 * u + 2
        stage1(t1, buf_b)
        stage2(t1 - 1, buf_a)

        @pl.when(t2 < _RP_PER_W)
        def _():
            stage1(t2, buf_a)

        stage2(t1, buf_b)
        return ()

    lax.fori_loop(0, _RP_PER_W // 2, pipe, ())

    pltpu.sync_copy(accv, out_hbm.at[wid])


def _sc_pair_loss(target_cls, dist, dmat):
    mesh = plsc.VectorSubcoreMesh(
        core_axis_name="c", subcore_axis_name="s", num_cores=2,
        num_subcores=16)
    f = pl.kernel(
        _sc_body,
        out_type=jax.ShapeDtypeStruct((_NW, _LANES), jnp.float32),
        mesh=mesh,
        compiler_params=pltpu.CompilerParams(needs_layout_passes=False),
        scratch_types=[
            pltpu.VMEM((_B,), jnp.int32),
            pltpu.VMEM((_CAP,), jnp.int32),
            pltpu.VMEM((_CAP,), jnp.float32),
            pltpu.VMEM((_CAP,), jnp.float32),
            pltpu.VMEM((_B,), jnp.float32),
            pltpu.VMEM((_B // 2,), jnp.float32),
            pltpu.VMEM((_CAP,), jnp.int32),
            pltpu.VMEM((_CAP,), jnp.float32),
            pltpu.VMEM((_CAP,), jnp.float32),
            pltpu.VMEM((_B,), jnp.float32),
            pltpu.VMEM((_B // 2,), jnp.float32),
            pltpu.VMEM((_LANES,), jnp.float32),
            pltpu.SemaphoreType.DMA,
            pltpu.SemaphoreType.DMA,
            pltpu.SemaphoreType.DMA,
            pltpu.SemaphoreType.DMA,
            pltpu.SemaphoreType.DMA,
            pltpu.SemaphoreType.DMA,
        ],
    )
    return f(target_cls, dist, dmat)


def kernel(output, target_cls, dmat):
    dmax = _dmat_max(dmat)
    dist = _dist_matrix(output, dmax)
    partials = _sc_pair_loss(target_cls, dist, dmat)
    return jnp.sum(partials) / (jnp.float32(_P) * dmax[0, 0])


# speedup vs baseline: 1.4761x; 1.1561x over previous
import jax
import jax.numpy as jnp
from jax import lax
from jax.experimental import pallas as pl
from jax.experimental.pallas import tpu as pltpu
from jax.experimental.pallas import tpu_sc as plsc

_M = 10000
_B = 4096
_D = 64
_P = _B * (_B - 1) // 2
_NW = 32
_RP_PER_W = (_B // 2) // _NW
_NG = _B // 16
_CAP = (_NG + 1) * 16
_LANES = 16


_NDM = _M * (_M - 1) // 2
_MAXCH = 4864 * 1024
_MAXG = _NDM // _MAXCH
_MAXTAIL = _NDM - _MAXG * _MAXCH


def _max_body(x_ref, tail_ref, o_ref):
    g = pl.program_id(0)

    @pl.when(g == 0)
    def _():
        o_ref[0, 0] = jnp.max(tail_ref[...])

    o_ref[0, 0] = jnp.maximum(o_ref[0, 0], jnp.max(x_ref[...]))


def _dmat_max(dmat):
    tail = lax.slice(dmat, (_MAXG * _MAXCH,), (_NDM,))
    return pl.pallas_call(
        _max_body,
        grid=(_MAXG,),
        in_specs=[
            pl.BlockSpec((_MAXCH,), lambda g: (g,)),
            pl.BlockSpec((_MAXTAIL,), lambda g: (0,)),
        ],
        out_specs=pl.BlockSpec(memory_space=pltpu.SMEM),
        out_shape=jax.ShapeDtypeStruct((1, 1), jnp.float32),
    )(dmat, tail)


_TILE = 512


def _sq_body(x_ref, ones_ref, sqc_ref, sqr_ref):
    xx = x_ref[...] * x_ref[...]
    sqc_ref[...] = jnp.sum(xx, axis=1, keepdims=True)
    sqr_ref[...] = lax.dot_general(
        ones_ref[...], xx, (((1,), (1,)), ((), ())),
        preferred_element_type=jnp.float32,
        precision=lax.Precision.HIGHEST,
    )


def _sq_norms(output):
    ones = jnp.ones((1, _D), jnp.float32)
    return pl.pallas_call(
        _sq_body,
        out_shape=(
            jax.ShapeDtypeStruct((_B, 1), jnp.float32),
            jax.ShapeDtypeStruct((1, _B), jnp.float32),
        ),
    )(output, ones)


def _dist_body(xi_ref, xj_ref, sqc_ref, sqr_ref, o_ref):
    g = lax.dot_general(
        xi_ref[...], xj_ref[...], (((1,), (1,)), ((), ())),
        preferred_element_type=jnp.float32,
        precision=lax.Precision.DEFAULT,
    )
    d2 = sqc_ref[...] + sqr_ref[...] - 2.0 * g
    o_ref[...] = jnp.sqrt(jnp.maximum(d2, 0.0))


def _dist_matrix(output):
    sqc, sqr = _sq_norms(output)
    nt = _B // _TILE
    return pl.pallas_call(
        _dist_body,
        grid=(nt, nt),
        in_specs=[
            pl.BlockSpec((_TILE, _D), lambda i, j: (i, 0)),
            pl.BlockSpec((_TILE, _D), lambda i, j: (j, 0)),
            pl.BlockSpec((_TILE, 1), lambda i, j: (i, 0)),
            pl.BlockSpec((1, _TILE), lambda i, j: (0, j)),
        ],
        out_specs=pl.BlockSpec((_TILE, _TILE), lambda i, j: (i, j)),
        out_shape=jax.ShapeDtypeStruct((_B, _B), jnp.float32),
    )(output, output, sqc, sqr)


def _sc_body(tc_hbm, d_hbm, dmat_hbm, outd_hbm, outg_hbm, outm_hbm,
             tcv,
             cidx_a, mask_a, tval_a, drow1_a, drow2_a,
             cidx_b, mask_b, tval_b, drow1_b, drow2_b,
             accd_v, accg_v, accm_v,
             sem_d1a, sem_d2a, sem_ga, sem_d1b, sem_d2b, sem_gb):
    wid = lax.axis_index("s") * 2 + lax.axis_index("c")
    row0 = wid * _RP_PER_W
    iota = lax.broadcasted_iota(jnp.int32, (_LANES,), 0)

    buf_a = (cidx_a, mask_a, tval_a, drow1_a, drow2_a,
             sem_d1a, sem_d2a, sem_ga)
    buf_b = (cidx_b, mask_b, tval_b, drow1_b, drow2_b,
             sem_d1b, sem_d2b, sem_gb)

    pltpu.sync_copy(tc_hbm, tcv)
    accd_v[...] = jnp.zeros((_LANES,), jnp.float32)
    accg_v[...] = jnp.zeros((_LANES,), jnp.float32)
    accm_v[...] = jnp.full((_LANES,), 2.0, jnp.float32)

    def zero_body(k, _):
        z = jnp.zeros((_LANES,), jnp.int32)
        cidx_a[pl.ds(k * _LANES, _LANES)] = z
        cidx_b[pl.ds(k * _LANES, _LANES)] = z
        return ()

    lax.fori_loop(0, _CAP // _LANES, zero_body, ())

    def row_geom(t):
        r1 = row0 + t
        r2 = (_B - 1) - r1
        g1 = lax.shift_right_logical(r1 + 1, 4)
        g2 = lax.shift_right_logical(r2 + 1, 4)
        return r1, r2, g1, g2

    def splat_tc(r):
        idx = jnp.full((_LANES,), r, jnp.int32)
        return plsc.load_gather(tcv, [idx])

    def cond_idx(ta16, tc16):
        i16 = jnp.minimum(ta16, tc16)
        j16 = jnp.maximum(ta16, tc16)
        neq = jnp.not_equal(i16, j16)
        ci = _M * i16 + j16 - lax.shift_right_logical(
            (i16 + 2) * (i16 + 1), 1)
        return jnp.where(neq, ci, 0), neq

    def stage1(t, buf):
        cidx_v, mask_v, tval_v, drow1, drow2, sem_d1, sem_d2, sem_g = buf
        r1, r2, g1, g2 = row_geom(t)
        pltpu.make_async_copy(d_hbm.at[r1], drow1, sem_d1).start()
        pltpu.make_async_copy(d_hbm.at[r2, pl.ds(_B // 2, _B // 2)], drow2,
                              sem_d2).start()

        ta1 = splat_tc(r1)
        ta2 = splat_tc(r2)
        o2 = (_NG - g1) * _LANES

        bvec = iota + g1 * _LANES
        tc16 = tcv[pl.ds(g1 * _LANES, _LANES)]
        ci, neq = cond_idx(ta1, tc16)
        m = jnp.logical_and(neq, bvec > r1)
        cidx_v[pl.ds(0, _LANES)] = ci
        mask_v[pl.ds(0, _LANES)] = jnp.where(m, 1.0, 0.0).astype(jnp.float32)

        def loop1(k, _):
            tck = tcv[pl.ds(k * _LANES, _LANES)]
            cik, neqk = cond_idx(ta1, tck)
            slot = (k - g1) * _LANES
            cidx_v[pl.ds(slot, _LANES)] = cik
            mask_v[pl.ds(slot, _LANES)] = jnp.where(
                neqk, 1.0, 0.0).astype(jnp.float32)
            return ()

        lax.fori_loop(g1 + 1, _NG, loop1, ())

        @pl.when(g2 < _NG)
        def _():
            bvec2 = iota + g2 * _LANES
            tcg = tcv[pl.ds(g2 * _LANES, _LANES)]
            ci2, neq2 = cond_idx(ta2, tcg)
            m2 = jnp.logical_and(neq2, bvec2 > r2)
            cidx_v[pl.ds(o2, _LANES)] = ci2
            mask_v[pl.ds(o2, _LANES)] = jnp.where(
                m2, 1.0, 0.0).astype(jnp.float32)

        def loop2(k, _):
            tck = tcv[pl.ds(k * _LANES, _LANES)]
            cik, neqk = cond_idx(ta2, tck)
            slot = o2 + (k - g2) * _LANES
            cidx_v[pl.ds(slot, _LANES)] = cik
            mask_v[pl.ds(slot, _LANES)] = jnp.where(
                neqk, 1.0, 0.0).astype(jnp.float32)
            return ()

        lax.fori_loop(jnp.minimum(g2 + 1, _NG), _NG, loop2, ())

        pltpu.make_async_copy(dmat_hbm.at[cidx_v], tval_v, sem_g).start()

    def stage2(t, buf):
        cidx_v, mask_v, tval_v, drow1, drow2, sem_d1, sem_d2, sem_g = buf
        r1, r2, g1, g2 = row_geom(t)
        o2 = (_NG - g1) * _LANES
        pltpu.make_async_copy(d_hbm.at[r1], drow1, sem_d1).wait()
        pltpu.make_async_copy(d_hbm.at[r2, pl.ds(_B // 2, _B // 2)], drow2,
                              sem_d2).wait()

        bvec = iota + g1 * _LANES
        d16 = drow1[pl.ds(g1 * _LANES, _LANES)]
        drow1[pl.ds(g1 * _LANES, _LANES)] = jnp.where(bvec > r1, d16, 0.0)

        @pl.when(g2 < _NG)
        def _():
            bvec2 = iota + g2 * _LANES
            off2 = g2 * _LANES - _B // 2
            d16b = drow2[pl.ds(off2, _LANES)]
            drow2[pl.ds(off2, _LANES)] = jnp.where(
                bvec2 > r2, d16b, 0.0)

        pltpu.make_async_copy(dmat_hbm.at[cidx_v], tval_v, sem_g).wait()

        def acc1(k, s):
            sd, sg, sm = s
            d = drow1[pl.ds(k * _LANES, _LANES)]
            slot = (k - g1) * _LANES
            tv = tval_v[pl.ds(slot, _LANES)]
            mv = mask_v[pl.ds(slot, _LANES)]
            return (sd + d, sg + tv * mv,
                    jnp.minimum(sm, jnp.where(mv > 0.5, d, 2.0)))

        z16 = jnp.zeros((_LANES,), jnp.float32)
        s1 = lax.fori_loop(g1, _NG, acc1,
                           (z16, z16, jnp.full((_LANES,), 2.0, jnp.float32)))

        def acc2(k, s):
            sd, sg, sm = s
            d = drow2[pl.ds(k * _LANES - _B // 2, _LANES)]
            slot = o2 + (k - g2) * _LANES
            tv = tval_v[pl.ds(slot, _LANES)]
            mv = mask_v[pl.ds(slot, _LANES)]
            return (sd + d, sg + tv * mv,
                    jnp.minimum(sm, jnp.where(mv > 0.5, d, 2.0)))

        sd2, sg2, sm2 = lax.fori_loop(jnp.minimum(g2, _NG), _NG, acc2, s1)
        accd_v[...] = accd_v[...] + sd2
        accg_v[...] = accg_v[...] + sg2
        accm_v[...] = jnp.minimum(accm_v[...], sm2)

    stage1(jnp.int32(0), buf_a)

    def pipe(u, _):
        t1 = 2 * u + 1
        t2 = 2 * u + 2
        stage1(t1, buf_b)
        stage2(t1 - 1, buf_a)

        @pl.when(t2 < _RP_PER_W)
        def _():
            stage1(t2, buf_a)

        stage2(t1, buf_b)
        return ()

    lax.fori_loop(0, _RP_PER_W // 2, pipe, ())

    pltpu.sync_copy(accd_v, outd_hbm.at[wid])
    pltpu.sync_copy(accg_v, outg_hbm.at[wid])
    pltpu.sync_copy(accm_v, outm_hbm.at[wid])


def _sc_pair_loss(target_cls, dist, dmat):
    mesh = plsc.VectorSubcoreMesh(
        core_axis_name="c", subcore_axis_name="s", num_cores=2,
        num_subcores=16)
    f = pl.kernel(
        _sc_body,
        out_type=(
            jax.ShapeDtypeStruct((_NW, _LANES), jnp.float32),
            jax.ShapeDtypeStruct((_NW, _LANES), jnp.float32),
            jax.ShapeDtypeStruct((_NW, _LANES), jnp.float32),
        ),
        mesh=mesh,
        compiler_params=pltpu.CompilerParams(needs_layout_passes=False),
        scratch_types=[
            pltpu.VMEM((_B,), jnp.int32),
            pltpu.VMEM((_CAP,), jnp.int32),
            pltpu.VMEM((_CAP,), jnp.float32),
            pltpu.VMEM((_CAP,), jnp.float32),
            pltpu.VMEM((_B,), jnp.float32),
            pltpu.VMEM((_B // 2,), jnp.float32),
            pltpu.VMEM((_CAP,), jnp.int32),
            pltpu.VMEM((_CAP,), jnp.float32),
            pltpu.VMEM((_CAP,), jnp.float32),
            pltpu.VMEM((_B,), jnp.float32),
            pltpu.VMEM((_B // 2,), jnp.float32),
            pltpu.VMEM((_LANES,), jnp.float32),
            pltpu.VMEM((_LANES,), jnp.float32),
            pltpu.VMEM((_LANES,), jnp.float32),
            pltpu.SemaphoreType.DMA,
            pltpu.SemaphoreType.DMA,
            pltpu.SemaphoreType.DMA,
            pltpu.SemaphoreType.DMA,
            pltpu.SemaphoreType.DMA,
            pltpu.SemaphoreType.DMA,
        ],
    )
    return f(target_cls, dist, dmat)


def _exact_loss(output, target_cls, dmat, dmax):
    iu = jnp.triu_indices(_B, k=1)
    pairs = target_cls[jnp.stack([iu[0], iu[1]])]
    sp = jnp.sort(pairs, axis=0)
    i, j = sp[0], sp[1]
    tgt_idx = _M * i + j - ((i + 2) * (i + 1)) // 2
    target = jnp.where(i == j, 0.0, dmat[tgt_idx] / dmax)
    sq = jnp.sum(output * output, axis=1)
    d2 = sq[:, None] + sq[None, :] - 2.0 * (output @ output.T)
    dist = jnp.sqrt(jnp.maximum(d2, 0.0) + jnp.eye(_B, dtype=output.dtype))
    d_sel = dist[iu[0], iu[1]]
    return jnp.mean(jnp.abs(d_sel - target))


def kernel(output, target_cls, dmat):
    dist = _dist_matrix(output)
    pd, pg, pm = _sc_pair_loss(target_cls, dist, dmat)
    dmax = _dmat_max(dmat)
    dmax_s = dmax[0, 0]
    sum_d = jnp.sum(pd)
    sum_g = jnp.sum(pg)
    dmin = jnp.min(pm)
    loss_linear = (sum_d - sum_g / dmax_s) / jnp.float32(_P)
    return lax.cond(
        dmin < 1.0,
        lambda: _exact_loss(output, target_cls, dmat, dmax_s),
        lambda: loss_linear,
    )


# speedup vs baseline: 1.5793x; 1.0699x over previous
import jax
import jax.numpy as jnp
from jax import lax
from jax.experimental import pallas as pl
from jax.experimental.pallas import tpu as pltpu
from jax.experimental.pallas import tpu_sc as plsc

_M = 10000
_B = 4096
_D = 64
_P = _B * (_B - 1) // 2
_NW = 32
_RP_PER_W = (_B // 2) // _NW
_NG = _B // 16
_CAP = (_NG + 1) * 16
_HCAP = 2048
_LANES = 16


_NDM = _M * (_M - 1) // 2
_MAXCH = 4864 * 1024
_MAXG = _NDM // _MAXCH
_MAXTAIL = _NDM - _MAXG * _MAXCH


def _max_body(x_ref, tail_ref, o_ref):
    g = pl.program_id(0)

    @pl.when(g == 0)
    def _():
        o_ref[0, 0] = jnp.max(tail_ref[...])

    o_ref[0, 0] = jnp.maximum(o_ref[0, 0], jnp.max(x_ref[...]))


def _dmat_max(dmat):
    tail = lax.slice(dmat, (_MAXG * _MAXCH,), (_NDM,))
    return pl.pallas_call(
        _max_body,
        grid=(_MAXG,),
        in_specs=[
            pl.BlockSpec((_MAXCH,), lambda g: (g,)),
            pl.BlockSpec((_MAXTAIL,), lambda g: (0,)),
        ],
        out_specs=pl.BlockSpec(memory_space=pltpu.SMEM),
        out_shape=jax.ShapeDtypeStruct((1, 1), jnp.float32),
    )(dmat, tail)


_TILE = 512


def _sq_body(x_ref, ones_ref, sqc_ref, sqr_ref):
    xx = x_ref[...] * x_ref[...]
    sqc_ref[...] = jnp.sum(xx, axis=1, keepdims=True)
    sqr_ref[...] = lax.dot_general(
        ones_ref[...], xx, (((1,), (1,)), ((), ())),
        preferred_element_type=jnp.float32,
        precision=lax.Precision.HIGHEST,
    )


def _sq_norms(output):
    ones = jnp.ones((1, _D), jnp.float32)
    return pl.pallas_call(
        _sq_body,
        out_shape=(
            jax.ShapeDtypeStruct((_B, 1), jnp.float32),
            jax.ShapeDtypeStruct((1, _B), jnp.float32),
        ),
    )(output, ones)


def _dist_body(xi_ref, xj_ref, sqc_ref, sqr_ref, o_ref):
    g = lax.dot_general(
        xi_ref[...], xj_ref[...], (((1,), (1,)), ((), ())),
        preferred_element_type=jnp.float32,
        precision=lax.Precision.DEFAULT,
    )
    d2 = sqc_ref[...] + sqr_ref[...] - 2.0 * g
    o_ref[...] = jnp.sqrt(jnp.maximum(d2, 0.0))


def _dist_matrix(output):
    sqc, sqr = _sq_norms(output)
    nt = _B // _TILE
    return pl.pallas_call(
        _dist_body,
        grid=(nt,),
        in_specs=[
            pl.BlockSpec((_TILE, _D), lambda i: (i, 0)),
            pl.BlockSpec((_B, _D), lambda i: (0, 0)),
            pl.BlockSpec((_TILE, 1), lambda i: (i, 0)),
            pl.BlockSpec((1, _B), lambda i: (0, 0)),
        ],
        out_specs=pl.BlockSpec((_TILE, _B), lambda i: (i, 0)),
        out_shape=jax.ShapeDtypeStruct((_B, _B), jnp.float32),
    )(output, output, sqc, sqr)


def _sc_body(tc_hbm, d_hbm, dmat_hbm, outd_hbm, outg_hbm, outm_hbm,
             tcv,
             cidx_a, mask_a, tval_a, drow1_a, drow2_a,
             cidx_b, mask_b, tval_b, drow1_b, drow2_b,
             accd_v, accg_v, accm_v,
             sem_d1a, sem_d2a, sem_ga, sem_d1b, sem_d2b, sem_gb):
    wid = lax.axis_index("s") * 2 + lax.axis_index("c")
    row0 = wid * _RP_PER_W
    iota = lax.broadcasted_iota(jnp.int32, (_LANES,), 0)

    buf_a = (cidx_a, mask_a, tval_a, drow1_a, drow2_a,
             sem_d1a, sem_d2a, sem_ga)
    buf_b = (cidx_b, mask_b, tval_b, drow1_b, drow2_b,
             sem_d1b, sem_d2b, sem_gb)

    pltpu.sync_copy(tc_hbm, tcv)
    accd_v[...] = jnp.zeros((_LANES,), jnp.float32)
    accg_v[...] = jnp.zeros((_LANES,), jnp.float32)
    accm_v[...] = jnp.full((_LANES,), 2.0, jnp.float32)

    def zero_body(k, _):
        z = jnp.zeros((_LANES,), jnp.int32)
        cidx_a[pl.ds(k * _LANES, _LANES)] = z
        cidx_b[pl.ds(k * _LANES, _LANES)] = z
        return ()

    lax.fori_loop(0, _CAP // _LANES, zero_body, ())

    def row_geom(t):
        r1 = row0 + t
        r2 = (_B - 1) - r1
        g1 = lax.shift_right_logical(r1 + 1, 4)
        g2 = lax.shift_right_logical(r2 + 1, 4)
        return r1, r2, g1, g2

    def splat_tc(r):
        idx = jnp.full((_LANES,), r, jnp.int32)
        return plsc.load_gather(tcv, [idx])

    def cond_idx(ta16, tc16):
        i16 = jnp.minimum(ta16, tc16)
        j16 = jnp.maximum(ta16, tc16)
        neq = jnp.not_equal(i16, j16)
        ci = _M * i16 + j16 - lax.shift_right_logical(
            (i16 + 2) * (i16 + 1), 1)
        return jnp.where(neq, ci, 0), neq

    def stage1(t, buf):
        cidx_v, mask_v, tval_v, drow1, drow2, sem_d1, sem_d2, sem_g = buf
        r1, r2, g1, g2 = row_geom(t)
        pltpu.make_async_copy(d_hbm.at[r1], drow1, sem_d1).start()
        pltpu.make_async_copy(d_hbm.at[r2, pl.ds(_B // 2, _B // 2)], drow2,
                              sem_d2).start()

        ta1 = splat_tc(r1)
        ta2 = splat_tc(r2)
        o2 = (_NG - g1) * _LANES

        bvec = iota + g1 * _LANES
        tc16 = tcv[pl.ds(g1 * _LANES, _LANES)]
        ci, neq = cond_idx(ta1, tc16)
        m = jnp.logical_and(neq, bvec > r1)
        cidx_v[pl.ds(0, _LANES)] = ci
        mask_v[pl.ds(0, _LANES)] = jnp.where(m, 1.0, 0.0).astype(jnp.float32)

        def loop1(k, _):
            tck = tcv[pl.ds(k * _LANES, _LANES)]
            cik, neqk = cond_idx(ta1, tck)
            slot = (k - g1) * _LANES
            cidx_v[pl.ds(slot, _LANES)] = cik
            mask_v[pl.ds(slot, _LANES)] = jnp.where(
                neqk, 1.0, 0.0).astype(jnp.float32)
            return ()

        lax.fori_loop(g1 + 1, g1 + 128, loop1, ())
        pltpu.make_async_copy(
            dmat_hbm.at[cidx_v.at[pl.ds(0, _HCAP)]],
            tval_v.at[pl.ds(0, _HCAP)], sem_g).start()
        lax.fori_loop(g1 + 128, _NG, loop1, ())

        @pl.when(g2 < _NG)
        def _():
            bvec2 = iota + g2 * _LANES
            tcg = tcv[pl.ds(g2 * _LANES, _LANES)]
            ci2, neq2 = cond_idx(ta2, tcg)
            m2 = jnp.logical_and(neq2, bvec2 > r2)
            cidx_v[pl.ds(o2, _LANES)] = ci2
            mask_v[pl.ds(o2, _LANES)] = jnp.where(
                m2, 1.0, 0.0).astype(jnp.float32)

        def loop2(k, _):
            tck = tcv[pl.ds(k * _LANES, _LANES)]
            cik, neqk = cond_idx(ta2, tck)
            slot = o2 + (k - g2) * _LANES
            cidx_v[pl.ds(slot, _LANES)] = cik
            mask_v[pl.ds(slot, _LANES)] = jnp.where(
                neqk, 1.0, 0.0).astype(jnp.float32)
            return ()

        lax.fori_loop(jnp.minimum(g2 + 1, _NG), _NG, loop2, ())

        pltpu.make_async_copy(
            dmat_hbm.at[cidx_v.at[pl.ds(_HCAP, _CAP - _HCAP)]],
            tval_v.at[pl.ds(_HCAP, _CAP - _HCAP)], sem_g).start()

    def stage2(t, buf):
        cidx_v, mask_v, tval_v, drow1, drow2, sem_d1, sem_d2, sem_g = buf
        r1, r2, g1, g2 = row_geom(t)
        o2 = (_NG - g1) * _LANES
        pltpu.make_async_copy(d_hbm.at[r1], drow1, sem_d1).wait()
        pltpu.make_async_copy(d_hbm.at[r2, pl.ds(_B // 2, _B // 2)], drow2,
                              sem_d2).wait()

        bvec = iota + g1 * _LANES
        d16 = drow1[pl.ds(g1 * _LANES, _LANES)]
        drow1[pl.ds(g1 * _LANES, _LANES)] = jnp.where(bvec > r1, d16, 0.0)

        @pl.when(g2 < _NG)
        def _():
            bvec2 = iota + g2 * _LANES
            off2 = g2 * _LANES - _B // 2
            d16b = drow2[pl.ds(off2, _LANES)]
            drow2[pl.ds(off2, _LANES)] = jnp.where(
                bvec2 > r2, d16b, 0.0)

        pltpu.make_async_copy(
            dmat_hbm.at[cidx_v.at[pl.ds(0, _HCAP)]],
            tval_v.at[pl.ds(0, _HCAP)], sem_g).wait()
        pltpu.make_async_copy(
            dmat_hbm.at[cidx_v.at[pl.ds(_HCAP, _CAP - _HCAP)]],
            tval_v.at[pl.ds(_HCAP, _CAP - _HCAP)], sem_g).wait()

        def acc1(k, s):
            sd, sg, sm = s
            d = drow1[pl.ds(k * _LANES, _LANES)]
            slot = (k - g1) * _LANES
            tv = tval_v[pl.ds(slot, _LANES)]
            mv = mask_v[pl.ds(slot, _LANES)]
            return (sd + d, sg + tv * mv,
                    jnp.minimum(sm, jnp.where(mv > 0.5, d, 2.0)))

        z16 = jnp.zeros((_LANES,), jnp.float32)
        s1 = lax.fori_loop(g1, _NG, acc1,
                           (z16, z16, jnp.full((_LANES,), 2.0, jnp.float32)))

        def acc2(k, s):
            sd, sg, sm = s
            d = drow2[pl.ds(k * _LANES - _B // 2, _LANES)]
            slot = o2 + (k - g2) * _LANES
            tv = tval_v[pl.ds(slot, _LANES)]
            mv = mask_v[pl.ds(slot, _LANES)]
            return (sd + d, sg + tv * mv,
                    jnp.minimum(sm, jnp.where(mv > 0.5, d, 2.0)))

        sd2, sg2, sm2 = lax.fori_loop(jnp.minimum(g2, _NG), _NG, acc2, s1)
        accd_v[...] = accd_v[...] + sd2
        accg_v[...] = accg_v[...] + sg2
        accm_v[...] = jnp.minimum(accm_v[...], sm2)

    stage1(jnp.int32(0), buf_a)

    def pipe(u, _):
        t1 = 2 * u + 1
        t2 = 2 * u + 2
        stage1(t1, buf_b)
        stage2(t1 - 1, buf_a)

        @pl.when(t2 < _RP_PER_W)
        def _():
            stage1(t2, buf_a)

        stage2(t1, buf_b)
        return ()

    lax.fori_loop(0, _RP_PER_W // 2, pipe, ())

    pltpu.sync_copy(accd_v, outd_hbm.at[wid])
    pltpu.sync_copy(accg_v, outg_hbm.at[wid])
    pltpu.sync_copy(accm_v, outm_hbm.at[wid])


def _sc_pair_loss(target_cls, dist, dmat):
    mesh = plsc.VectorSubcoreMesh(
        core_axis_name="c", subcore_axis_name="s", num_cores=2,
        num_subcores=16)
    f = pl.kernel(
        _sc_body,
        out_type=(
            jax.ShapeDtypeStruct((_NW, _LANES), jnp.float32),
            jax.ShapeDtypeStruct((_NW, _LANES), jnp.float32),
            jax.ShapeDtypeStruct((_NW, _LANES), jnp.float32),
        ),
        mesh=mesh,
        compiler_params=pltpu.CompilerParams(needs_layout_passes=False),
        scratch_types=[
            pltpu.VMEM((_B,), jnp.int32),
            pltpu.VMEM((_CAP,), jnp.int32),
            pltpu.VMEM((_CAP,), jnp.float32),
            pltpu.VMEM((_CAP,), jnp.float32),
            pltpu.VMEM((_B,), jnp.float32),
            pltpu.VMEM((_B // 2,), jnp.float32),
            pltpu.VMEM((_CAP,), jnp.int32),
            pltpu.VMEM((_CAP,), jnp.float32),
            pltpu.VMEM((_CAP,), jnp.float32),
            pltpu.VMEM((_B,), jnp.float32),
            pltpu.VMEM((_B // 2,), jnp.float32),
            pltpu.VMEM((_LANES,), jnp.float32),
            pltpu.VMEM((_LANES,), jnp.float32),
            pltpu.VMEM((_LANES,), jnp.float32),
            pltpu.SemaphoreType.DMA,
            pltpu.SemaphoreType.DMA,
            pltpu.SemaphoreType.DMA,
            pltpu.SemaphoreType.DMA,
            pltpu.SemaphoreType.DMA,
            pltpu.SemaphoreType.DMA,
        ],
    )
    return f(target_cls, dist, dmat)


def _exact_loss(output, target_cls, dmat, dmax):
    iu = jnp.triu_indices(_B, k=1)
    pairs = target_cls[jnp.stack([iu[0], iu[1]])]
    sp = jnp.sort(pairs, axis=0)
    i, j = sp[0], sp[1]
    tgt_idx = _M * i + j - ((i + 2) * (i + 1)) // 2
    target = jnp.where(i == j, 0.0, dmat[tgt_idx] / dmax)
    sq = jnp.sum(output * output, axis=1)
    d2 = sq[:, None] + sq[None, :] - 2.0 * (output @ output.T)
    dist = jnp.sqrt(jnp.maximum(d2, 0.0) + jnp.eye(_B, dtype=output.dtype))
    d_sel = dist[iu[0], iu[1]]
    return jnp.mean(jnp.abs(d_sel - target))


def kernel(output, target_cls, dmat):
    dist = _dist_matrix(output)
    pd, pg, pm = _sc_pair_loss(target_cls, dist, dmat)
    dmax = _dmat_max(dmat)
    dmax_s = dmax[0, 0]
    sum_d = jnp.sum(pd)
    sum_g = jnp.sum(pg)
    dmin = jnp.min(pm)
    loss_linear = (sum_d - sum_g / dmax_s) / jnp.float32(_P)
    return lax.cond(
        dmin < 1.0,
        lambda: _exact_loss(output, target_cls, dmat, dmax_s),
        lambda: loss_linear,
    )
